# Initial kernel scaffold; baseline (speedup 1.0000x reference)
import jax
import jax.numpy as jnp
from jax.experimental import pallas as pl


def kernel(x, edge_index, cluster_ids, clusterid_inbatch, W1, W2):
    raise NotImplementedError("write your pallas kernel here")


# speedup vs baseline: 75.3638x; 75.3638x over previous
import functools

import jax
import jax.numpy as jnp
from jax import lax
from jax.experimental import pallas as pl
from jax.experimental.pallas import tpu as pltpu
from jax.experimental.pallas import tpu_sc as plsc

N_NODES = 10000
N_EDGES = 320000
D_FEAT = 128
EMBED = 64
N_CLUSTERS = 64
FPAD = 128

NC = 2
NS = 16
LANES = 16
NW = NC * NS
EPW = N_EDGES // NW
NCHUNK = EPW // LANES
PAD_ROWS = 10240
ROWS_PT = PAD_ROWS // NS
DUMMY = N_NODES
RBLK = N_NODES // 10


def _sc_edge_pass(feat, src, dst, cid, cib16):
    mesh = plsc.VectorSubcoreMesh(core_axis_name="c", subcore_axis_name="s")

    @functools.partial(
        pl.kernel,
        mesh=mesh,
        compiler_params=pltpu.CompilerParams(needs_layout_passes=False),
        out_type=jax.ShapeDtypeStruct((NC, PAD_ROWS, FPAD), jnp.float32),
        scratch_types=[
            pltpu.VMEM((EPW + LANES,), jnp.int32),
            pltpu.VMEM((EPW + LANES,), jnp.int32),
            pltpu.VMEM((N_NODES,), jnp.int32),
            pltpu.VMEM((16,), jnp.int32),
            pltpu.VMEM((N_CLUSTERS,), jnp.int32),
            pltpu.VMEM((LANES, FPAD), jnp.float32),
            pltpu.VMEM_SHARED((PAD_ROWS, FPAD), jnp.float32),
            pltpu.SemaphoreType.DMA,
        ],
    )
    def body(feat_h, src_h, dst_h, cid_h, cib_h, agg_out,
             src_v, dst_v, cid_v, cib_v, selt_v, rows_v, agg_sh, sem):
        c = lax.axis_index("c")
        s = lax.axis_index("s")
        w = c * NS + s
        i16 = lax.iota(jnp.int32, LANES)
        zf16 = jnp.zeros((LANES,), jnp.float32)

        pltpu.sync_copy(src_h.at[pl.ds(w * EPW, EPW)], src_v.at[pl.ds(0, EPW)])
        pltpu.sync_copy(dst_h.at[pl.ds(w * EPW, EPW)], dst_v.at[pl.ds(0, EPW)])
        pltpu.sync_copy(cid_h, cid_v)
        pltpu.sync_copy(cib_h, cib_v)

        for k in range(N_CLUSTERS // LANES):
            selt_v[pl.ds(k * LANES, LANES)] = jnp.zeros((LANES,), jnp.int32)
        bvals = cib_v[...]
        plsc.store_scatter(
            selt_v,
            [jnp.clip(bvals, 0, N_CLUSTERS - 1)],
            jnp.ones((LANES,), jnp.int32),
            mask=(bvals >= 0) & (bvals < N_CLUSTERS),
        )

        for r in range(LANES):
            for q in range(FPAD // LANES):
                rows_v[r, pl.ds(q * LANES, LANES)] = zf16

        def zrow_body(j, carry):
            pltpu.sync_copy(
                rows_v, agg_sh.at[pl.ds(s * ROWS_PT + j * LANES, LANES), :])
            return carry
        lax.fori_loop(0, ROWS_PT // LANES, zrow_body, 0)

        def scan_body(i, cnt):
            off = i * LANES
            sv = src_v[pl.ds(off, LANES)]
            dv = dst_v[pl.ds(off, LANES)]
            cs = plsc.load_gather(cid_v, [sv])
            cd = plsc.load_gather(cid_v, [dv])
            slv = plsc.load_gather(selt_v, [cd])
            m = (cs == cd) & (slv == 1)
            mi = m.astype(jnp.int32)
            pos = jnp.maximum(cnt + jnp.cumsum(mi) - 1, 0)
            plsc.store_scatter(src_v, [pos], sv, mask=m)
            plsc.store_scatter(dst_v, [pos], dv, mask=m)
            return cnt + jnp.sum(mi)

        cnt = lax.fori_loop(0, NCHUNK, scan_body, jnp.int32(0))

        plsc.store_scatter(src_v, [cnt + i16], jnp.zeros((LANES,), jnp.int32))
        plsc.store_scatter(dst_v, [cnt + i16],
                           jnp.full((LANES,), DUMMY, jnp.int32))

        plsc.subcore_barrier()

        def agg_body(i, carry):
            idx = i * LANES + i16
            sv = plsc.load_gather(src_v, [idx])
            dv = plsc.load_gather(dst_v, [idx])
            pltpu.async_copy(feat_h.at[sv], rows_v, sem).wait()
            pltpu.sync_copy(rows_v, agg_sh.at[dv], add=True)
            return carry

        nch = (cnt + LANES - 1) // LANES
        lax.fori_loop(0, nch, agg_body, 0)

        plsc.subcore_barrier()

        pltpu.sync_copy(agg_sh.at[pl.ds(s * ROWS_PT, ROWS_PT), :],
                        agg_out.at[c, pl.ds(s * ROWS_PT, ROWS_PT), :])

    return body(feat, src, dst, cid, cib16)


def _tc_in_matmul(x, W1p):
    def mm(x_ref, w_ref, o_ref):
        m = lax.dot_general(
            x_ref[...], w_ref[...], (((1,), (0,)), ((), ())),
            preferred_element_type=jnp.float32)
        col = lax.broadcasted_iota(jnp.int32, (RBLK, FPAD), 1)
        o_ref[...] = m + jnp.where(col == EMBED, 1.0, 0.0)

    return pl.pallas_call(
        mm,
        grid=(10,),
        in_specs=[pl.BlockSpec((RBLK, D_FEAT), lambda i: (i, 0)),
                  pl.BlockSpec((D_FEAT, FPAD), lambda i: (0, 0))],
        out_specs=pl.BlockSpec((RBLK, FPAD), lambda i: (i, 0)),
        out_shape=jax.ShapeDtypeStruct((N_NODES, FPAD), jnp.float32),
    )(x, W1p)


def _tc_mid(agg_a, agg_b, W2p):
    def mid(aa_ref, ab_ref, w_ref, o_ref):
        a = aa_ref[...] + ab_ref[...]
        col = lax.broadcasted_iota(jnp.int32, (RBLK, FPAD), 1)
        d = jnp.sum(jnp.where(col == EMBED, a, 0.0), axis=1, keepdims=True)
        inv = 1.0 / jnp.maximum(d, 1.0)
        h = jnp.maximum(a, 0.0) * inv
        z = lax.dot_general(
            h, w_ref[...], (((1,), (0,)), ((), ())),
            preferred_element_type=jnp.float32)
        o_ref[...] = z + jnp.where(col == EMBED, 1.0, 0.0)

    return pl.pallas_call(
        mid,
        grid=(10,),
        in_specs=[pl.BlockSpec((RBLK, FPAD), lambda i: (i, 0)),
                  pl.BlockSpec((RBLK, FPAD), lambda i: (i, 0)),
                  pl.BlockSpec((FPAD, FPAD), lambda i: (0, 0))],
        out_specs=pl.BlockSpec((RBLK, FPAD), lambda i: (i, 0)),
        out_shape=jax.ShapeDtypeStruct((N_NODES, FPAD), jnp.float32),
    )(agg_a, agg_b, W2p)


def _tc_final(agg_a, agg_b):
    def fin(aa_ref, ab_ref, o_ref):
        a = aa_ref[...] + ab_ref[...]
        col = lax.broadcasted_iota(jnp.int32, (RBLK, FPAD), 1)
        d = jnp.sum(jnp.where(col == EMBED, a, 0.0), axis=1, keepdims=True)
        inv = 1.0 / jnp.maximum(d, 1.0)
        o_ref[...] = a * inv

    return pl.pallas_call(
        fin,
        grid=(10,),
        in_specs=[pl.BlockSpec((RBLK, FPAD), lambda i: (i, 0)),
                  pl.BlockSpec((RBLK, FPAD), lambda i: (i, 0))],
        out_specs=pl.BlockSpec((RBLK, FPAD), lambda i: (i, 0)),
        out_shape=jax.ShapeDtypeStruct((N_NODES, FPAD), jnp.float32),
    )(agg_a, agg_b)


def kernel(x, edge_index, cluster_ids, clusterid_inbatch, W1, W2):
    src = edge_index[0].astype(jnp.int32)
    dst = edge_index[1].astype(jnp.int32)
    cid = cluster_ids.astype(jnp.int32)
    cib16 = jnp.pad(clusterid_inbatch.astype(jnp.int32),
                    (0, 16 - clusterid_inbatch.shape[0]), constant_values=-1)
    W1p = jnp.pad(W1, ((0, 0), (0, FPAD - EMBED)))
    W2p = jnp.pad(W2, ((0, FPAD - EMBED), (0, FPAD - EMBED)))

    y = _tc_in_matmul(x, W1p)
    agg1 = _sc_edge_pass(y, src, dst, cid, cib16)
    z = _tc_mid(agg1[0, :N_NODES], agg1[1, :N_NODES], W2p)
    agg2 = _sc_edge_pass(z, src, dst, cid, cib16)
    out_full = _tc_final(agg2[0, :N_NODES], agg2[1, :N_NODES])
    return out_full[:, :EMBED]


# speedup vs baseline: 84.7168x; 1.1241x over previous
import functools

import jax
import jax.numpy as jnp
from jax import lax
from jax.experimental import pallas as pl
from jax.experimental.pallas import tpu as pltpu
from jax.experimental.pallas import tpu_sc as plsc

N_NODES = 10000
N_EDGES = 320000
D_FEAT = 128
EMBED = 64
N_CLUSTERS = 64
FPAD = 128

NC = 2
NS = 16
LANES = 16
NW = NC * NS
EPW = N_EDGES // NW
NCHUNK = EPW // LANES
PAD_ROWS = 10240
ROWS_PT = PAD_ROWS // NS
DUMMY = N_NODES
RBLK = N_NODES // 10


def _sc_edge_pass(feat, src, dst, cid, cib16):
    mesh = plsc.VectorSubcoreMesh(core_axis_name="c", subcore_axis_name="s")

    @functools.partial(
        pl.kernel,
        mesh=mesh,
        compiler_params=pltpu.CompilerParams(needs_layout_passes=False),
        out_type=jax.ShapeDtypeStruct((NC, PAD_ROWS, FPAD), jnp.float32),
        scratch_types=[
            pltpu.VMEM((EPW + LANES,), jnp.int32),
            pltpu.VMEM((EPW + LANES,), jnp.int32),
            pltpu.VMEM((N_NODES,), jnp.int32),
            pltpu.VMEM((16,), jnp.int32),
            pltpu.VMEM((N_CLUSTERS,), jnp.int32),
            pltpu.VMEM((LANES, FPAD), jnp.float32),
            pltpu.VMEM_SHARED((PAD_ROWS, FPAD), jnp.float32),
            pltpu.SemaphoreType.DMA,
            pltpu.SemaphoreType.DMA,
        ],
    )
    def body(feat_h, src_h, dst_h, cid_h, cib_h, agg_out,
             src_v, dst_v, cid_v, cib_v, selt_v, rows_v, agg_sh, sem, zsem):
        c = lax.axis_index("c")
        s = lax.axis_index("s")
        w = c * NS + s
        i16 = lax.iota(jnp.int32, LANES)
        zf16 = jnp.zeros((LANES,), jnp.float32)

        stage = [
            pltpu.async_copy(src_h.at[pl.ds(w * EPW, EPW)],
                             src_v.at[pl.ds(0, EPW)], sem),
            pltpu.async_copy(dst_h.at[pl.ds(w * EPW, EPW)],
                             dst_v.at[pl.ds(0, EPW)], sem),
            pltpu.async_copy(cid_h, cid_v, sem),
            pltpu.async_copy(cib_h, cib_v, sem),
        ]

        for r in range(LANES):
            for q in range(FPAD // LANES):
                rows_v[r, pl.ds(q * LANES, LANES)] = zf16
        zcopies = [
            pltpu.async_copy(
                rows_v, agg_sh.at[pl.ds(s * ROWS_PT + j * LANES, LANES), :],
                zsem)
            for j in range(ROWS_PT // LANES)
        ]
        for d in stage:
            d.wait()

        for k in range(N_CLUSTERS // LANES):
            selt_v[pl.ds(k * LANES, LANES)] = jnp.zeros((LANES,), jnp.int32)
        bvals = cib_v[...]
        plsc.store_scatter(
            selt_v,
            [jnp.clip(bvals, 0, N_CLUSTERS - 1)],
            jnp.ones((LANES,), jnp.int32),
            mask=(bvals >= 0) & (bvals < N_CLUSTERS),
        )

        UNROLL = 5
        def scan_body(i, cnt):
            for j in range(UNROLL):
                off = (i * UNROLL + j) * LANES
                sv = src_v[pl.ds(off, LANES)]
                dv = dst_v[pl.ds(off, LANES)]
                cs = plsc.load_gather(cid_v, [sv])
                cd = plsc.load_gather(cid_v, [dv])
                slv = plsc.load_gather(selt_v, [cd])
                m = (cs == cd) & (slv == 1)
                plsc.store_compressed(src_v.at[pl.ds(cnt, LANES)], sv, mask=m)
                plsc.store_compressed(dst_v.at[pl.ds(cnt, LANES)], dv, mask=m)
                cnt = cnt + jnp.sum(m.astype(jnp.int32))
            return cnt

        cnt = lax.fori_loop(0, NCHUNK // UNROLL, scan_body, jnp.int32(0))

        for d in zcopies:
            d.wait()

        plsc.store_scatter(src_v, [cnt + i16], jnp.zeros((LANES,), jnp.int32))
        plsc.store_scatter(dst_v, [cnt + i16],
                           jnp.full((LANES,), DUMMY, jnp.int32))

        plsc.subcore_barrier()

        def agg_body(i, carry):
            idx = i * LANES + i16
            sv = plsc.load_gather(src_v, [idx])
            dv = plsc.load_gather(dst_v, [idx])
            pltpu.async_copy(feat_h.at[sv], rows_v, sem).wait()
            pltpu.sync_copy(rows_v, agg_sh.at[dv], add=True)
            return carry

        nch = (cnt + LANES - 1) // LANES
        lax.fori_loop(0, nch, agg_body, 0)

        plsc.subcore_barrier()

        pltpu.sync_copy(agg_sh.at[pl.ds(s * ROWS_PT, ROWS_PT), :],
                        agg_out.at[c, pl.ds(s * ROWS_PT, ROWS_PT), :])

    return body(feat, src, dst, cid, cib16)


def _tc_in_matmul(x, W1p):
    def mm(x_ref, w_ref, o_ref):
        m = lax.dot_general(
            x_ref[...], w_ref[...], (((1,), (0,)), ((), ())),
            preferred_element_type=jnp.float32)
        col = lax.broadcasted_iota(jnp.int32, (RBLK, FPAD), 1)
        o_ref[...] = m + jnp.where(col == EMBED, 1.0, 0.0)

    return pl.pallas_call(
        mm,
        grid=(10,),
        in_specs=[pl.BlockSpec((RBLK, D_FEAT), lambda i: (i, 0)),
                  pl.BlockSpec((D_FEAT, FPAD), lambda i: (0, 0))],
        out_specs=pl.BlockSpec((RBLK, FPAD), lambda i: (i, 0)),
        out_shape=jax.ShapeDtypeStruct((N_NODES, FPAD), jnp.float32),
    )(x, W1p)


def _tc_mid(agg_a, agg_b, W2p):
    def mid(aa_ref, ab_ref, w_ref, o_ref):
        a = aa_ref[...] + ab_ref[...]
        col = lax.broadcasted_iota(jnp.int32, (RBLK, FPAD), 1)
        d = jnp.sum(jnp.where(col == EMBED, a, 0.0), axis=1, keepdims=True)
        inv = 1.0 / jnp.maximum(d, 1.0)
        h = jnp.maximum(a, 0.0) * inv
        z = lax.dot_general(
            h, w_ref[...], (((1,), (0,)), ((), ())),
            preferred_element_type=jnp.float32)
        o_ref[...] = z + jnp.where(col == EMBED, 1.0, 0.0)

    return pl.pallas_call(
        mid,
        grid=(10,),
        in_specs=[pl.BlockSpec((RBLK, FPAD), lambda i: (i, 0)),
                  pl.BlockSpec((RBLK, FPAD), lambda i: (i, 0)),
                  pl.BlockSpec((FPAD, FPAD), lambda i: (0, 0))],
        out_specs=pl.BlockSpec((RBLK, FPAD), lambda i: (i, 0)),
        out_shape=jax.ShapeDtypeStruct((N_NODES, FPAD), jnp.float32),
    )(agg_a, agg_b, W2p)


def _tc_final(agg_a, agg_b):
    def fin(aa_ref, ab_ref, o_ref):
        a = aa_ref[...] + ab_ref[...]
        col = lax.broadcasted_iota(jnp.int32, (RBLK, FPAD), 1)
        d = jnp.sum(jnp.where(col == EMBED, a, 0.0), axis=1, keepdims=True)
        inv = 1.0 / jnp.maximum(d, 1.0)
        o_ref[...] = a * inv

    return pl.pallas_call(
        fin,
        grid=(10,),
        in_specs=[pl.BlockSpec((RBLK, FPAD), lambda i: (i, 0)),
                  pl.BlockSpec((RBLK, FPAD), lambda i: (i, 0))],
        out_specs=pl.BlockSpec((RBLK, FPAD), lambda i: (i, 0)),
        out_shape=jax.ShapeDtypeStruct((N_NODES, FPAD), jnp.float32),
    )(agg_a, agg_b)


def kernel(x, edge_index, cluster_ids, clusterid_inbatch, W1, W2):
    src = edge_index[0].astype(jnp.int32)
    dst = edge_index[1].astype(jnp.int32)
    cid = cluster_ids.astype(jnp.int32)
    cib16 = jnp.pad(clusterid_inbatch.astype(jnp.int32),
                    (0, 16 - clusterid_inbatch.shape[0]), constant_values=-1)
    W1p = jnp.pad(W1, ((0, 0), (0, FPAD - EMBED)))
    W2p = jnp.pad(W2, ((0, FPAD - EMBED), (0, FPAD - EMBED)))

    y = _tc_in_matmul(x, W1p)
    agg1 = _sc_edge_pass(y, src, dst, cid, cib16)
    z = _tc_mid(agg1[0, :N_NODES], agg1[1, :N_NODES], W2p)
    agg2 = _sc_edge_pass(z, src, dst, cid, cib16)
    out_full = _tc_final(agg2[0, :N_NODES], agg2[1, :N_NODES])
    return out_full[:, :EMBED]


# speedup vs baseline: 100.0688x; 1.1812x over previous
import functools

import jax
import jax.numpy as jnp
from jax import lax
from jax.experimental import pallas as pl
from jax.experimental.pallas import tpu as pltpu
from jax.experimental.pallas import tpu_sc as plsc

N_NODES = 10000
N_EDGES = 320000
D_FEAT = 128
EMBED = 64
N_CLUSTERS = 64
FPAD = 128

NC = 2
NS = 16
LANES = 16
NW = NC * NS
EPW = N_EDGES // NW
NCHUNK = EPW // LANES
ESTAGE = -(-EPW // 128) * 128
PAD_ROWS = 10240
ROWS_PT = PAD_ROWS // NS
DUMMY = N_NODES
ZROWS = 64
RBLK = N_NODES // 10


def _sc_edge_pass(feat, edge_index, cid, cib16):
    mesh = plsc.VectorSubcoreMesh(core_axis_name="c", subcore_axis_name="s")

    @functools.partial(
        pl.kernel,
        mesh=mesh,
        compiler_params=pltpu.CompilerParams(needs_layout_passes=False),
        out_type=jax.ShapeDtypeStruct((NC, PAD_ROWS, FPAD), jnp.float32),
        scratch_types=[
            pltpu.VMEM((EPW + LANES,), jnp.int32),
            pltpu.VMEM((EPW + LANES,), jnp.int32),
            pltpu.VMEM((N_NODES,), jnp.int32),
            pltpu.VMEM((16,), jnp.int32),
            pltpu.VMEM((N_CLUSTERS,), jnp.int32),
            pltpu.VMEM((LANES, FPAD), jnp.float32),
            pltpu.VMEM((ZROWS, FPAD), jnp.float32),
            pltpu.VMEM_SHARED((PAD_ROWS, FPAD), jnp.float32),
            pltpu.SemaphoreType.DMA,
            pltpu.SemaphoreType.DMA,
        ],
    )
    def body(feat_h, ei_h, cid_h, cib_h, agg_out,
             src_v, dst_v, cid_v, cib_v, selt_v, rows_v, zblk_v, agg_sh,
             sem, zsem):
        c = lax.axis_index("c")
        s = lax.axis_index("s")
        w = c * NS + s
        i16 = lax.iota(jnp.int32, LANES)
        zf16 = jnp.zeros((LANES,), jnp.float32)

        stage = [
            pltpu.async_copy(ei_h.at[pl.ds(w * EPW, EPW)],
                             src_v.at[pl.ds(0, EPW)], sem),
            pltpu.async_copy(ei_h.at[pl.ds(N_EDGES + w * EPW, EPW)],
                             dst_v.at[pl.ds(0, EPW)], sem),
            pltpu.async_copy(cid_h, cid_v, sem),
            pltpu.async_copy(cib_h, cib_v, sem),
        ]

        for r in range(ZROWS):
            for q in range(FPAD // LANES):
                zblk_v[r, pl.ds(q * LANES, LANES)] = zf16
        zcopies = [
            pltpu.async_copy(
                zblk_v, agg_sh.at[pl.ds(s * ROWS_PT + j * ZROWS, ZROWS), :],
                zsem)
            for j in range(ROWS_PT // ZROWS)
        ]
        for d in stage:
            d.wait()

        for k in range(N_CLUSTERS // LANES):
            selt_v[pl.ds(k * LANES, LANES)] = jnp.zeros((LANES,), jnp.int32)
        bvals = cib_v[...]
        plsc.store_scatter(
            selt_v,
            [jnp.clip(bvals, 0, N_CLUSTERS - 1)],
            jnp.ones((LANES,), jnp.int32),
            mask=(bvals >= 0) & (bvals < N_CLUSTERS),
        )

        UNROLL = 5
        def scan_body(i, cnt):
            for j in range(UNROLL):
                off = (i * UNROLL + j) * LANES
                sv = src_v[pl.ds(off, LANES)]
                dv = dst_v[pl.ds(off, LANES)]
                cs = plsc.load_gather(cid_v, [sv])
                cd = plsc.load_gather(cid_v, [dv])
                slv = plsc.load_gather(selt_v, [cd])
                m = (cs == cd) & (slv == 1)
                plsc.store_compressed(src_v.at[pl.ds(cnt, LANES)], sv, mask=m)
                plsc.store_compressed(dst_v.at[pl.ds(cnt, LANES)], dv, mask=m)
                cnt = cnt + jnp.sum(m.astype(jnp.int32))
            return cnt

        cnt = lax.fori_loop(0, NCHUNK // UNROLL, scan_body, jnp.int32(0))

        for d in zcopies:
            d.wait()

        plsc.store_scatter(src_v, [cnt + i16], jnp.zeros((LANES,), jnp.int32))
        plsc.store_scatter(dst_v, [cnt + i16],
                           jnp.full((LANES,), DUMMY, jnp.int32))

        plsc.subcore_barrier()

        def agg_body(i, carry):
            idx = i * LANES + i16
            sv = plsc.load_gather(src_v, [idx])
            dv = plsc.load_gather(dst_v, [idx])
            pltpu.async_copy(feat_h.at[sv], rows_v, sem).wait()
            pltpu.sync_copy(rows_v, agg_sh.at[dv], add=True)
            return carry

        nch = (cnt + LANES - 1) // LANES
        lax.fori_loop(0, nch, agg_body, 0)

        plsc.subcore_barrier()

        pltpu.sync_copy(agg_sh.at[pl.ds(s * ROWS_PT, ROWS_PT), :],
                        agg_out.at[c, pl.ds(s * ROWS_PT, ROWS_PT), :])

    return body(feat, edge_index, cid, cib16)


def _tc_in_matmul(x, W1p):
    def mm(x_ref, w_ref, o_ref):
        m = lax.dot_general(
            x_ref[...], w_ref[...], (((1,), (0,)), ((), ())),
            preferred_element_type=jnp.float32)
        col = lax.broadcasted_iota(jnp.int32, (RBLK, FPAD), 1)
        o_ref[...] = m + jnp.where(col == EMBED, 1.0, 0.0)

    return pl.pallas_call(
        mm,
        grid=(10,),
        in_specs=[pl.BlockSpec((RBLK, D_FEAT), lambda i: (i, 0)),
                  pl.BlockSpec((D_FEAT, FPAD), lambda i: (0, 0))],
        out_specs=pl.BlockSpec((RBLK, FPAD), lambda i: (i, 0)),
        out_shape=jax.ShapeDtypeStruct((N_NODES, FPAD), jnp.float32),
    )(x, W1p)


def _tc_mid(agg, W2p):
    def mid(aa_ref, ab_ref, w_ref, o_ref):
        a = aa_ref[0] + ab_ref[0]
        col = lax.broadcasted_iota(jnp.int32, (RBLK, FPAD), 1)
        d = jnp.sum(jnp.where(col == EMBED, a, 0.0), axis=1, keepdims=True)
        inv = 1.0 / jnp.maximum(d, 1.0)
        h = jnp.maximum(a, 0.0) * inv
        z = lax.dot_general(
            h, w_ref[...], (((1,), (0,)), ((), ())),
            preferred_element_type=jnp.float32)
        o_ref[...] = z + jnp.where(col == EMBED, 1.0, 0.0)

    return pl.pallas_call(
        mid,
        grid=(10,),
        in_specs=[pl.BlockSpec((1, RBLK, FPAD), lambda i: (0, i, 0)),
                  pl.BlockSpec((1, RBLK, FPAD), lambda i: (1, i, 0)),
                  pl.BlockSpec((FPAD, FPAD), lambda i: (0, 0))],
        out_specs=pl.BlockSpec((RBLK, FPAD), lambda i: (i, 0)),
        out_shape=jax.ShapeDtypeStruct((N_NODES, FPAD), jnp.float32),
    )(agg, agg, W2p)


def _tc_final(agg):
    def fin(aa_ref, ab_ref, o_ref):
        a = aa_ref[0] + ab_ref[0]
        col = lax.broadcasted_iota(jnp.int32, (RBLK, FPAD), 1)
        d = jnp.sum(jnp.where(col == EMBED, a, 0.0), axis=1, keepdims=True)
        inv = 1.0 / jnp.maximum(d, 1.0)
        o_ref[...] = a * inv

    return pl.pallas_call(
        fin,
        grid=(10,),
        in_specs=[pl.BlockSpec((1, RBLK, FPAD), lambda i: (0, i, 0)),
                  pl.BlockSpec((1, RBLK, FPAD), lambda i: (1, i, 0))],
        out_specs=pl.BlockSpec((RBLK, FPAD), lambda i: (i, 0)),
        out_shape=jax.ShapeDtypeStruct((N_NODES, FPAD), jnp.float32),
    )(agg, agg)


def kernel(x, edge_index, cluster_ids, clusterid_inbatch, W1, W2):
    ei = edge_index.astype(jnp.int32).reshape(2 * N_EDGES)
    cid = cluster_ids.astype(jnp.int32)
    cib16 = jnp.pad(clusterid_inbatch.astype(jnp.int32),
                    (0, 16 - clusterid_inbatch.shape[0]), constant_values=-1)
    W1p = jnp.pad(W1, ((0, 0), (0, FPAD - EMBED)))
    W2p = jnp.pad(W2, ((0, FPAD - EMBED), (0, FPAD - EMBED)))

    y = _tc_in_matmul(x, W1p)
    agg1 = _sc_edge_pass(y, ei, cid, cib16)
    z = _tc_mid(agg1, W2p)
    agg2 = _sc_edge_pass(z, ei, cid, cib16)
    out_full = _tc_final(agg2)
    return out_full[:, :EMBED]


# speedup vs baseline: 117.5929x; 1.1751x over previous
import functools

import jax
import jax.numpy as jnp
from jax import lax
from jax.experimental import pallas as pl
from jax.experimental.pallas import tpu as pltpu
from jax.experimental.pallas import tpu_sc as plsc

N_NODES = 10000
N_EDGES = 320000
D_FEAT = 128
EMBED = 64
N_CLUSTERS = 64
FPAD = 128

NC = 2
NS = 16
LANES = 16
NW = NC * NS
EPW = N_EDGES // NW
NCHUNK = EPW // LANES
PAD_ROWS = 10240
ROWS_PT = PAD_ROWS // NS
DUMMY = N_NODES
ZROWS = 64
RBLK = N_NODES // 10


def _sc_edge_pass(feat, ei, cid, cib16, with_deg):
    mesh = plsc.VectorSubcoreMesh(core_axis_name="c", subcore_axis_name="s")

    agg_t = jax.ShapeDtypeStruct((NC, PAD_ROWS, FPAD), jnp.float32)
    deg_t = jax.ShapeDtypeStruct((NC, PAD_ROWS), jnp.float32)
    deg_scratch = [
        pltpu.VMEM((ROWS_PT,), jnp.float32),
        pltpu.VMEM((LANES,), jnp.float32),
        pltpu.VMEM_SHARED((PAD_ROWS,), jnp.float32),
    ] if with_deg else []

    @functools.partial(
        pl.kernel,
        mesh=mesh,
        compiler_params=pltpu.CompilerParams(needs_layout_passes=False),
        out_type=(agg_t, deg_t) if with_deg else agg_t,
        scratch_types=[
            pltpu.VMEM((EPW + LANES,), jnp.int32),
            pltpu.VMEM((EPW + LANES,), jnp.int32),
            pltpu.VMEM((N_NODES,), jnp.int32),
            pltpu.VMEM((16,), jnp.int32),
            pltpu.VMEM((N_CLUSTERS,), jnp.int32),
            pltpu.VMEM((LANES, FPAD), jnp.float32),
            pltpu.VMEM((ZROWS, FPAD), jnp.float32),
            pltpu.VMEM_SHARED((PAD_ROWS, FPAD), jnp.float32),
            *deg_scratch,
            pltpu.SemaphoreType.DMA,
            pltpu.SemaphoreType.DMA,
        ],
    )
    def body(feat_h, ei_h, cid_h, cib_h, *outs_scratch):
        if with_deg:
            (agg_out, deg_out, src_v, dst_v, cid_v, cib_v, selt_v, rows_v,
             zblk_v, agg_sh, zdeg_v, ones_v, deg_sh, sem, zsem) = outs_scratch
        else:
            (agg_out, src_v, dst_v, cid_v, cib_v, selt_v, rows_v,
             zblk_v, agg_sh, sem, zsem) = outs_scratch
        c = lax.axis_index("c")
        s = lax.axis_index("s")
        w = c * NS + s
        i16 = lax.iota(jnp.int32, LANES)
        zf16 = jnp.zeros((LANES,), jnp.float32)

        stage = [
            pltpu.async_copy(ei_h.at[pl.ds(w * EPW, EPW)],
                             src_v.at[pl.ds(0, EPW)], sem),
            pltpu.async_copy(ei_h.at[pl.ds(N_EDGES + w * EPW, EPW)],
                             dst_v.at[pl.ds(0, EPW)], sem),
            pltpu.async_copy(cid_h, cid_v, sem),
            pltpu.async_copy(cib_h, cib_v, sem),
        ]

        for r in range(ZROWS):
            for q in range(FPAD // LANES):
                zblk_v[r, pl.ds(q * LANES, LANES)] = zf16
        zcopies = [
            pltpu.async_copy(
                zblk_v, agg_sh.at[pl.ds(s * ROWS_PT + j * ZROWS, ZROWS), :],
                zsem)
            for j in range(ROWS_PT // ZROWS)
        ]
        if with_deg:
            for j in range(ROWS_PT // LANES):
                zdeg_v[pl.ds(j * LANES, LANES)] = zf16
            ones_v[...] = jnp.ones((LANES,), jnp.float32)
            zcopies.append(pltpu.async_copy(
                zdeg_v, deg_sh.at[pl.ds(s * ROWS_PT, ROWS_PT)], zsem))
        for d in stage:
            d.wait()

        for k in range(N_CLUSTERS // LANES):
            selt_v[pl.ds(k * LANES, LANES)] = jnp.zeros((LANES,), jnp.int32)
        bvals = cib_v[...]
        plsc.store_scatter(
            selt_v,
            [jnp.clip(bvals, 0, N_CLUSTERS - 1)],
            jnp.ones((LANES,), jnp.int32),
            mask=(bvals >= 0) & (bvals < N_CLUSTERS),
        )

        UNROLL = 5
        def scan_body(i, cnt):
            svs, dvs, ms, sums = [], [], [], []
            for j in range(UNROLL):
                off = (i * UNROLL + j) * LANES
                sv = src_v[pl.ds(off, LANES)]
                dv = dst_v[pl.ds(off, LANES)]
                cs = plsc.load_gather(cid_v, [sv])
                cd = plsc.load_gather(cid_v, [dv])
                slv = plsc.load_gather(selt_v, [cd])
                m = (cs == cd) & (slv == 1)
                svs.append(sv)
                dvs.append(dv)
                ms.append(m)
                sums.append(jnp.sum(m.astype(jnp.int32)))
            offs = [cnt]
            for j in range(UNROLL):
                offs.append(offs[-1] + sums[j])
            for j in range(UNROLL):
                plsc.store_compressed(src_v.at[pl.ds(offs[j], LANES)],
                                      svs[j], mask=ms[j])
                plsc.store_compressed(dst_v.at[pl.ds(offs[j], LANES)],
                                      dvs[j], mask=ms[j])
            return offs[-1]

        cnt = lax.fori_loop(0, NCHUNK // UNROLL, scan_body, jnp.int32(0))

        for d in zcopies:
            d.wait()

        plsc.store_scatter(src_v, [cnt + i16], jnp.zeros((LANES,), jnp.int32))
        plsc.store_scatter(dst_v, [cnt + i16],
                           jnp.full((LANES,), DUMMY, jnp.int32))

        plsc.subcore_barrier()

        def agg_body(i, carry):
            idx = i * LANES + i16
            sv = plsc.load_gather(src_v, [idx])
            dv = plsc.load_gather(dst_v, [idx])
            pltpu.async_copy(feat_h.at[sv], rows_v, sem).wait()
            pltpu.sync_copy(rows_v, agg_sh.at[dv], add=True)
            if with_deg:
                pltpu.sync_copy(ones_v, deg_sh.at[dv], add=True)
            return carry

        nch = (cnt + LANES - 1) // LANES
        lax.fori_loop(0, nch, agg_body, 0)

        plsc.subcore_barrier()

        pltpu.sync_copy(agg_sh.at[pl.ds(s * ROWS_PT, ROWS_PT), :],
                        agg_out.at[c, pl.ds(s * ROWS_PT, ROWS_PT), :])
        if with_deg:
            pltpu.sync_copy(deg_sh.at[pl.ds(s * ROWS_PT, ROWS_PT)],
                            deg_out.at[c, pl.ds(s * ROWS_PT, ROWS_PT)])

    return body(feat, ei, cid, cib16)


def _tc_mid(aggx, deg3, W1p, W2p):
    def mid(aa_ref, ab_ref, da_ref, db_ref, w1_ref, w2_ref, o_ref):
        a = aa_ref[0] + ab_ref[0]
        d = da_ref[0] + db_ref[0]
        inv = 1.0 / jnp.maximum(d, 1.0)
        h = lax.dot_general(
            a * inv, w1_ref[...], (((1,), (0,)), ((), ())),
            preferred_element_type=jnp.float32)
        o_ref[...] = lax.dot_general(
            jnp.maximum(h, 0.0), w2_ref[...], (((1,), (0,)), ((), ())),
            preferred_element_type=jnp.float32)

    return pl.pallas_call(
        mid,
        grid=(10,),
        in_specs=[pl.BlockSpec((1, RBLK, FPAD), lambda i: (0, i, 0)),
                  pl.BlockSpec((1, RBLK, FPAD), lambda i: (1, i, 0)),
                  pl.BlockSpec((1, RBLK, 1), lambda i: (0, i, 0)),
                  pl.BlockSpec((1, RBLK, 1), lambda i: (1, i, 0)),
                  pl.BlockSpec((FPAD, FPAD), lambda i: (0, 0)),
                  pl.BlockSpec((FPAD, FPAD), lambda i: (0, 0))],
        out_specs=pl.BlockSpec((RBLK, FPAD), lambda i: (i, 0)),
        out_shape=jax.ShapeDtypeStruct((N_NODES, FPAD), jnp.float32),
    )(aggx, aggx, deg3, deg3, W1p, W2p)


def _tc_final(agg2, deg3):
    def fin(aa_ref, ab_ref, da_ref, db_ref, o_ref):
        a = aa_ref[0] + ab_ref[0]
        d = da_ref[0] + db_ref[0]
        inv = 1.0 / jnp.maximum(d, 1.0)
        val = (a * inv)[:, :EMBED]
        o_ref[...] = lax.transpose(val, (1, 0))

    return pl.pallas_call(
        fin,
        grid=(1,),
        in_specs=[pl.BlockSpec((1, N_NODES, FPAD), lambda i: (0, 0, 0)),
                  pl.BlockSpec((1, N_NODES, FPAD), lambda i: (1, 0, 0)),
                  pl.BlockSpec((1, N_NODES, 1), lambda i: (0, 0, 0)),
                  pl.BlockSpec((1, N_NODES, 1), lambda i: (1, 0, 0))],
        out_specs=pl.BlockSpec((EMBED, N_NODES), lambda i: (0, 0)),
        out_shape=jax.ShapeDtypeStruct((EMBED, N_NODES), jnp.float32),
    )(agg2, agg2, deg3, deg3)


def kernel(x, edge_index, cluster_ids, clusterid_inbatch, W1, W2):
    ei = edge_index.astype(jnp.int32).reshape(2 * N_EDGES)
    cid = cluster_ids.astype(jnp.int32)
    cib16 = jnp.pad(clusterid_inbatch.astype(jnp.int32),
                    (0, 16 - clusterid_inbatch.shape[0]), constant_values=-1)
    W1p = jnp.pad(W1, ((0, 0), (0, FPAD - EMBED)))
    W2p = jnp.pad(W2, ((0, FPAD - EMBED), (0, FPAD - EMBED)))

    aggx, deg = _sc_edge_pass(x, ei, cid, cib16, True)
    deg3 = deg.reshape(NC, PAD_ROWS, 1)
    z = _tc_mid(aggx, deg3, W1p, W2p)
    agg2 = _sc_edge_pass(z, ei, cid, cib16, False)
    return _tc_final(agg2, deg3).T


# speedup vs baseline: 127.9444x; 1.0880x over previous
import functools

import jax
import jax.numpy as jnp
from jax import lax
from jax.experimental import pallas as pl
from jax.experimental.pallas import tpu as pltpu
from jax.experimental.pallas import tpu_sc as plsc

N_NODES = 10000
N_EDGES = 320000
D_FEAT = 128
EMBED = 64
N_CLUSTERS = 64
FPAD = 128

NC = 2
NS = 16
LANES = 16
NW = NC * NS
EPW = N_EDGES // NW
NCHUNK = EPW // LANES
PAD_ROWS = 10240
ROWS_PT = PAD_ROWS // NS
DUMMY = N_NODES
ZROWS = 64
RBLK = PAD_ROWS // 10


def _sc_edge_pass(feat, ei, cid, cib16, with_deg):
    mesh = plsc.VectorSubcoreMesh(core_axis_name="c", subcore_axis_name="s")

    agg_t = jax.ShapeDtypeStruct((NC, PAD_ROWS, FPAD), jnp.float32)
    deg_t = jax.ShapeDtypeStruct((NC, PAD_ROWS), jnp.float32)
    deg_scratch = [
        pltpu.VMEM((ROWS_PT,), jnp.float32),
        pltpu.VMEM((LANES,), jnp.float32),
        pltpu.VMEM_SHARED((PAD_ROWS,), jnp.float32),
    ] if with_deg else []

    @functools.partial(
        pl.kernel,
        mesh=mesh,
        compiler_params=pltpu.CompilerParams(needs_layout_passes=False),
        out_type=(agg_t, deg_t) if with_deg else agg_t,
        scratch_types=[
            pltpu.VMEM((EPW + LANES,), jnp.int32),
            pltpu.VMEM((EPW + LANES,), jnp.int32),
            pltpu.VMEM((N_NODES,), jnp.int32),
            pltpu.VMEM((16,), jnp.int32),
            pltpu.VMEM((N_CLUSTERS,), jnp.int32),
            pltpu.VMEM((LANES, FPAD), jnp.float32),
            pltpu.VMEM((ZROWS, FPAD), jnp.float32),
            pltpu.VMEM_SHARED((PAD_ROWS, FPAD), jnp.float32),
            *deg_scratch,
            pltpu.SemaphoreType.DMA,
            pltpu.SemaphoreType.DMA,
        ],
    )
    def body(feat_h, ei_h, cid_h, cib_h, *outs_scratch):
        if with_deg:
            (agg_out, deg_out, src_v, dst_v, cid_v, cib_v, selt_v, rows_v,
             zblk_v, agg_sh, zdeg_v, ones_v, deg_sh, sem, zsem) = outs_scratch
        else:
            (agg_out, src_v, dst_v, cid_v, cib_v, selt_v, rows_v,
             zblk_v, agg_sh, sem, zsem) = outs_scratch
        c = lax.axis_index("c")
        s = lax.axis_index("s")
        w = c * NS + s
        i16 = lax.iota(jnp.int32, LANES)
        zf16 = jnp.zeros((LANES,), jnp.float32)

        stage = [
            pltpu.async_copy(ei_h.at[pl.ds(w * EPW, EPW)],
                             src_v.at[pl.ds(0, EPW)], sem),
            pltpu.async_copy(ei_h.at[pl.ds(N_EDGES + w * EPW, EPW)],
                             dst_v.at[pl.ds(0, EPW)], sem),
            pltpu.async_copy(cid_h, cid_v, sem),
            pltpu.async_copy(cib_h, cib_v, sem),
        ]

        for r in range(ZROWS):
            for q in range(FPAD // LANES):
                zblk_v[r, pl.ds(q * LANES, LANES)] = zf16
        zcopies = [
            pltpu.async_copy(
                zblk_v, agg_sh.at[pl.ds(s * ROWS_PT + j * ZROWS, ZROWS), :],
                zsem)
            for j in range(ROWS_PT // ZROWS)
        ]
        if with_deg:
            for j in range(ROWS_PT // LANES):
                zdeg_v[pl.ds(j * LANES, LANES)] = zf16
            ones_v[...] = jnp.ones((LANES,), jnp.float32)
            zcopies.append(pltpu.async_copy(
                zdeg_v, deg_sh.at[pl.ds(s * ROWS_PT, ROWS_PT)], zsem))
        for d in stage:
            d.wait()

        for k in range(N_CLUSTERS // LANES):
            selt_v[pl.ds(k * LANES, LANES)] = jnp.zeros((LANES,), jnp.int32)
        bvals = cib_v[...]
        plsc.store_scatter(
            selt_v,
            [jnp.clip(bvals, 0, N_CLUSTERS - 1)],
            jnp.ones((LANES,), jnp.int32),
            mask=(bvals >= 0) & (bvals < N_CLUSTERS),
        )

        UNROLL = 5
        def scan_body(i, cnt):
            svs, dvs, ms, sums = [], [], [], []
            for j in range(UNROLL):
                off = (i * UNROLL + j) * LANES
                sv = src_v[pl.ds(off, LANES)]
                dv = dst_v[pl.ds(off, LANES)]
                cs = plsc.load_gather(cid_v, [sv])
                cd = plsc.load_gather(cid_v, [dv])
                slv = plsc.load_gather(selt_v, [cd])
                m = (cs == cd) & (slv == 1)
                svs.append(sv)
                dvs.append(dv)
                ms.append(m)
                sums.append(jnp.sum(m.astype(jnp.int32)))
            offs = [cnt]
            for j in range(UNROLL):
                offs.append(offs[-1] + sums[j])
            for j in range(UNROLL):
                plsc.store_compressed(src_v.at[pl.ds(offs[j], LANES)],
                                      svs[j], mask=ms[j])
                plsc.store_compressed(dst_v.at[pl.ds(offs[j], LANES)],
                                      dvs[j], mask=ms[j])
            return offs[-1]

        cnt = lax.fori_loop(0, NCHUNK // UNROLL, scan_body, jnp.int32(0))

        for d in zcopies:
            d.wait()

        plsc.store_scatter(src_v, [cnt + i16], jnp.zeros((LANES,), jnp.int32))
        plsc.store_scatter(dst_v, [cnt + i16],
                           jnp.full((LANES,), DUMMY, jnp.int32))

        plsc.subcore_barrier()

        def agg_body(i, carry):
            idx = i * LANES + i16
            sv = plsc.load_gather(src_v, [idx])
            dv = plsc.load_gather(dst_v, [idx])
            pltpu.async_copy(feat_h.at[sv], rows_v, sem).wait()
            pltpu.sync_copy(rows_v, agg_sh.at[dv], add=True)
            if with_deg:
                pltpu.sync_copy(ones_v, deg_sh.at[dv], add=True)
            return carry

        nch = (cnt + LANES - 1) // LANES
        lax.fori_loop(0, nch, agg_body, 0)

        plsc.subcore_barrier()

        pltpu.sync_copy(agg_sh.at[pl.ds(s * ROWS_PT, ROWS_PT), :],
                        agg_out.at[c, pl.ds(s * ROWS_PT, ROWS_PT), :])
        if with_deg:
            pltpu.sync_copy(deg_sh.at[pl.ds(s * ROWS_PT, ROWS_PT)],
                            deg_out.at[c, pl.ds(s * ROWS_PT, ROWS_PT)])

    return body(feat, ei, cid, cib16)


def _tc_mid(aggx, deg, W1p, W2p):
    def mid(aa_ref, ab_ref, d_ref, w1_ref, w2_ref, o_ref):
        i = pl.program_id(0)
        a = aa_ref[0] + ab_ref[0]
        drow = d_ref[0, pl.ds(i * RBLK, RBLK)] + d_ref[1, pl.ds(i * RBLK, RBLK)]
        d = lax.transpose(drow.reshape(1, RBLK), (1, 0))
        inv = 1.0 / jnp.maximum(d, 1.0)
        h = lax.dot_general(
            a * inv, w1_ref[...], (((1,), (0,)), ((), ())),
            preferred_element_type=jnp.float32)
        o_ref[...] = lax.dot_general(
            jnp.maximum(h, 0.0), w2_ref[...], (((1,), (0,)), ((), ())),
            preferred_element_type=jnp.float32)

    return pl.pallas_call(
        mid,
        grid=(10,),
        in_specs=[pl.BlockSpec((1, RBLK, FPAD), lambda i: (0, i, 0)),
                  pl.BlockSpec((1, RBLK, FPAD), lambda i: (1, i, 0)),
                  pl.BlockSpec((NC, PAD_ROWS), lambda i: (0, 0)),
                  pl.BlockSpec((FPAD, FPAD), lambda i: (0, 0)),
                  pl.BlockSpec((FPAD, FPAD), lambda i: (0, 0))],
        out_specs=pl.BlockSpec((RBLK, FPAD), lambda i: (i, 0)),
        out_shape=jax.ShapeDtypeStruct((PAD_ROWS, FPAD), jnp.float32),
    )(aggx, aggx, deg, W1p, W2p)


def _tc_final(agg2, deg):
    def fin(aa_ref, ab_ref, d_ref, o_ref):
        a = aa_ref[0] + ab_ref[0]
        drow = d_ref[0] + d_ref[1]
        d = lax.transpose(drow.reshape(1, PAD_ROWS), (1, 0))
        inv = 1.0 / jnp.maximum(d, 1.0)
        val = (a * inv)[:N_NODES, :EMBED]
        o_ref[...] = lax.transpose(val, (1, 0))

    return pl.pallas_call(
        fin,
        grid=(1,),
        in_specs=[pl.BlockSpec((1, PAD_ROWS, FPAD), lambda i: (0, 0, 0)),
                  pl.BlockSpec((1, PAD_ROWS, FPAD), lambda i: (1, 0, 0)),
                  pl.BlockSpec((NC, PAD_ROWS), lambda i: (0, 0))],
        out_specs=pl.BlockSpec((EMBED, N_NODES), lambda i: (0, 0)),
        out_shape=jax.ShapeDtypeStruct((EMBED, N_NODES), jnp.float32),
    )(agg2, agg2, deg)


def kernel(x, edge_index, cluster_ids, clusterid_inbatch, W1, W2):
    ei = edge_index.astype(jnp.int32).reshape(2 * N_EDGES)
    cid = cluster_ids.astype(jnp.int32)
    cib16 = jnp.pad(clusterid_inbatch.astype(jnp.int32),
                    (0, 16 - clusterid_inbatch.shape[0]), constant_values=-1)
    W1p = jnp.pad(W1, ((0, 0), (0, FPAD - EMBED)))
    W2p = jnp.pad(W2, ((0, FPAD - EMBED), (0, FPAD - EMBED)))

    aggx, deg = _sc_edge_pass(x, ei, cid, cib16, True)
    z = _tc_mid(aggx, deg, W1p, W2p)
    agg2 = _sc_edge_pass(z, ei, cid, cib16, False)
    return _tc_final(agg2, deg).T
